# baseline (speedup 1.0000x reference)
import functools

import jax
import jax.numpy as jnp
from jax import lax
from jax.experimental import pallas as pl
from jax.experimental.pallas import tpu as pltpu
from jax.experimental.pallas import tpu_sc as plsc

_BLK = 64
_CHUNK = 512
_NC, _NS = 2, 16
_NW = _NC * _NS


def _gelu_exact(v):
    return 0.5 * v * (1.0 + lax.erf(v * 0.7071067811865476))


def _router_body(x_ref, wr_ref, tpp_ref, gidx_ref, prob_ref, info_ref, aux_ref):
    x = x_ref[...]
    wr = wr_ref[...]
    T = x.shape[0]
    E = wr.shape[1]
    Tpad = gidx_ref.shape[1]

    logits = jnp.dot(x, wr, preferred_element_type=jnp.float32)
    m = jnp.max(logits, axis=1, keepdims=True)
    p = jnp.exp(logits - m)
    probs = p / jnp.sum(p, axis=1, keepdims=True)
    top_prob = jnp.max(probs, axis=1, keepdims=True)
    e_iota = lax.broadcasted_iota(jnp.int32, (T, E), 1)
    top_idx = jnp.min(jnp.where(probs >= top_prob, e_iota, E),
                      axis=1, keepdims=True)
    onehot = (top_idx == e_iota).astype(jnp.float32)

    ones_col = jnp.ones((T, 1), jnp.float32)
    sizes_col = lax.dot_general(onehot, ones_col, (((0,), (0,)), ((), ())),
                                preferred_element_type=jnp.float32)
    ps_col = jnp.floor((sizes_col + (_BLK - 1)) * (1.0 / _BLK)) * _BLK
    ei = lax.broadcasted_iota(jnp.int32, (E, E), 0)
    ej = lax.broadcasted_iota(jnp.int32, (E, E), 1)
    lstrict = (ej < ei).astype(jnp.float32)
    lincl = (ej <= ei).astype(jnp.float32)
    po_col = jnp.dot(lstrict, ps_col, preferred_element_type=jnp.float32)
    tot_col = jnp.dot(lincl, ps_col, preferred_element_type=jnp.float32)

    CH = 256
    rloc = lax.broadcasted_iota(jnp.int32, (CH, CH), 0)
    cloc = lax.broadcasted_iota(jnp.int32, (CH, CH), 1)
    ltri = (cloc < rloc).astype(jnp.float32)
    carry = jnp.zeros((1, E), jnp.float32)
    ranks = []
    for k in range(T // CH):
        oh_k = onehot[k * CH:(k + 1) * CH]
        cs_k = jnp.dot(ltri, oh_k, preferred_element_type=jnp.float32) + carry
        ranks.append(jnp.sum(cs_k * oh_k, axis=1, keepdims=True))
        carry = carry + jnp.sum(oh_k, axis=0, keepdims=True)
    rank = jnp.concatenate(ranks, axis=0)

    po_t = jnp.dot(onehot, po_col, preferred_element_type=jnp.float32)
    tpp_i = (po_t + rank).astype(jnp.int32)
    tpp_ref[...] = tpp_i

    tvec = lax.broadcasted_iota(jnp.int32, (T, 1), 0)
    t_hi = (tvec >> 3).astype(jnp.float32)
    t_lo = (tvec & 7).astype(jnp.float32)
    p_hi = top_prob.astype(jnp.bfloat16).astype(jnp.float32)
    p_lo = top_prob - p_hi
    lhs = jnp.concatenate([t_hi, t_lo, p_hi, p_lo], axis=1)
    for k in range(Tpad // _CHUNK):
        pp_iota = lax.broadcasted_iota(jnp.int32, (T, _CHUNK), 1) + (k * _CHUNK)
        mk = (tpp_i == pp_iota).astype(jnp.float32)
        acc = lax.dot_general(lhs, mk, (((0,), (0,)), ((), ())),
                              preferred_element_type=jnp.float32)
        gi_k = acc[0:1] * 8.0 + acc[1:2]
        pr_k = acc[2:3] + acc[3:4]
        pp_row = lax.broadcasted_iota(jnp.int32, (1, _CHUNK), 1) + (k * _CHUNK)
        fill = pp_row - jnp.where(pp_row >= T, T, 0)
        gi_i = jnp.where(pr_k > 0.0, gi_k.astype(jnp.int32), fill)
        gidx_ref[:, pl.ds(k * _CHUNK, _CHUNK)] = gi_i
        prob_ref[:, pl.ds(k * _CHUNK, _CHUNK)] = pr_k

    nlane = info_ref.shape[1]
    ti = lax.broadcasted_iota(jnp.int32, (1, nlane), 1).astype(jnp.float32) * float(_BLK)
    cmp = (ti >= tot_col).astype(jnp.int32)
    te = jnp.minimum(jnp.sum(cmp, axis=0, keepdims=True), E - 1)
    total = jnp.sum(ps_col)
    used = (ti < total).astype(jnp.int32)
    info_ref[0:1, :] = te
    info_ref[1:2, :] = used

    imp = jnp.sum(probs, axis=0, keepdims=True) / float(T)
    sizes_row = jnp.sum(onehot, axis=0, keepdims=True)
    aux = float(E) * jnp.sum(imp * sizes_row / float(T))
    aux_ref[...] = jnp.zeros_like(aux_ref) + aux


def _router(x_flat, Wr, Tpad):
    T, C = x_flat.shape
    E = Wr.shape[1]
    return pl.pallas_call(
        _router_body,
        out_shape=[
            jax.ShapeDtypeStruct((T, 1), jnp.int32),
            jax.ShapeDtypeStruct((1, Tpad), jnp.int32),
            jax.ShapeDtypeStruct((1, Tpad), jnp.float32),
            jax.ShapeDtypeStruct((8, 128), jnp.int32),
            jax.ShapeDtypeStruct((1, 128), jnp.float32),
        ],
    )(x_flat, Wr)


def _ffn_body(info_ref, xp_ref, pp_ref, w1_ref, b1_ref, w2_ref, b2_ref,
              y_ref):
    i = pl.program_id(0)

    @pl.when(info_ref[1, i] == 1)
    def _():
        xb = xp_ref[...]
        h = jnp.dot(xb, w1_ref[0], preferred_element_type=jnp.float32)
        h = _gelu_exact(h + b1_ref[0])
        yb = jnp.dot(h, w2_ref[0], preferred_element_type=jnp.float32)
        y_ref[...] = (yb + b2_ref[0]) * pp_ref[0]


def _ffn(info, x_padded, prob3, W1, b1r, W2, b2r):
    Tpad, C = x_padded.shape
    E, _, H = W1.shape
    ntiles = Tpad // _BLK
    grid_spec = pltpu.PrefetchScalarGridSpec(
        num_scalar_prefetch=1,
        grid=(ntiles,),
        in_specs=[
            pl.BlockSpec((_BLK, C), lambda i, nfo: (i, 0)),
            pl.BlockSpec((1, _BLK, 1), lambda i, nfo: (i, 0, 0)),
            pl.BlockSpec((1, C, H), lambda i, nfo: (nfo[0, i], 0, 0)),
            pl.BlockSpec((1, 1, H), lambda i, nfo: (nfo[0, i], 0, 0)),
            pl.BlockSpec((1, H, C), lambda i, nfo: (nfo[0, i], 0, 0)),
            pl.BlockSpec((1, 1, C), lambda i, nfo: (nfo[0, i], 0, 0)),
        ],
        out_specs=pl.BlockSpec((_BLK, C), lambda i, nfo: (i, 0)),
    )
    return pl.pallas_call(
        _ffn_body,
        grid_spec=grid_spec,
        out_shape=jax.ShapeDtypeStruct((Tpad, C), jnp.float32),
    )(info, x_padded, prob3, W1, b1r, W2, b2r)


def _sc_gather(table, idx):
    n = idx.shape[0]
    C = table.shape[1]
    rpw = n // _NW
    mesh = plsc.VectorSubcoreMesh(core_axis_name="c", subcore_axis_name="s")

    @functools.partial(
        pl.kernel, mesh=mesh,
        out_type=jax.ShapeDtypeStruct((n, C), jnp.float32),
        scratch_types=[
            pltpu.VMEM((rpw,), jnp.int32),
            pltpu.VMEM((rpw, C), jnp.float32),
            pltpu.SemaphoreType.DMA,
        ],
    )
    def k(table_hbm, idx_hbm, out_hbm, idx_v, rows_v, sem):
        wid = lax.axis_index("s") * _NC + lax.axis_index("c")
        base = wid * rpw
        pltpu.sync_copy(idx_hbm.at[pl.ds(base, rpw)], idx_v)
        pltpu.async_copy(table_hbm.at[idx_v], rows_v, sem).wait()
        pltpu.sync_copy(rows_v, out_hbm.at[pl.ds(base, rpw)])

    return k(table, idx)


def _sc_dispatch(x_flat, gidx):
    return _sc_gather(x_flat, gidx)


def _sc_combine(y_scaled, tpp):
    return _sc_gather(y_scaled, tpp)


def kernel(x, Wr, W1, b1, W2, b2):
    b, t, c = x.shape
    e, _, hdim = W1.shape
    x_flat = x.reshape(t, c)
    Tpad = t + e * _BLK
    ntiles = Tpad // _BLK

    tpp2d, gidx2d, prob2d, info, aux2d = _router(x_flat, Wr, Tpad)
    tpp = tpp2d.reshape(t)
    gidx = gidx2d.reshape(Tpad)
    prob3 = prob2d.reshape(ntiles, _BLK, 1)

    x_padded = _sc_dispatch(x_flat, gidx)
    b1r = b1.reshape(e, 1, hdim)
    b2r = b2.reshape(e, 1, c)
    y_scaled = _ffn(info, x_padded, prob3, W1, b1r, W2, b2r)
    y_flat = _sc_combine(y_scaled, tpp)
    return y_flat.reshape(b, t, c), aux2d[0, 0]


# speedup vs baseline: 1.2555x; 1.2555x over previous
import functools

import jax
import jax.numpy as jnp
from jax import lax
from jax.experimental import pallas as pl
from jax.experimental.pallas import tpu as pltpu
from jax.experimental.pallas import tpu_sc as plsc

_BLK = 256
_CHUNK = 512
_NC, _NS = 2, 16
_NW = _NC * _NS


def _gelu_exact(v):
    return 0.5 * v * (1.0 + lax.erf(v * 0.7071067811865476))


def _router_body(x_ref, wr_ref, tpp_ref, gidx_ref, prob_ref, info_ref, aux_ref):
    x = x_ref[...]
    wr = wr_ref[...]
    T = x.shape[0]
    E = wr.shape[1]
    Tpad = gidx_ref.shape[1]

    logits = jnp.dot(x, wr, preferred_element_type=jnp.float32)
    m = jnp.max(logits, axis=1, keepdims=True)
    p = jnp.exp(logits - m)
    probs = p / jnp.sum(p, axis=1, keepdims=True)
    top_prob = jnp.max(probs, axis=1, keepdims=True)
    e_iota = lax.broadcasted_iota(jnp.int32, (T, E), 1)
    top_idx = jnp.min(jnp.where(probs >= top_prob, e_iota, E),
                      axis=1, keepdims=True)
    onehot = (top_idx == e_iota).astype(jnp.float32)

    ones_col = jnp.ones((T, 1), jnp.float32)
    sizes_col = lax.dot_general(onehot, ones_col, (((0,), (0,)), ((), ())),
                                preferred_element_type=jnp.float32)
    ps_col = jnp.floor((sizes_col + (_BLK - 1)) * (1.0 / _BLK)) * _BLK
    ei = lax.broadcasted_iota(jnp.int32, (E, E), 0)
    ej = lax.broadcasted_iota(jnp.int32, (E, E), 1)
    lstrict = (ej < ei).astype(jnp.float32)
    lincl = (ej <= ei).astype(jnp.float32)
    po_col = jnp.dot(lstrict, ps_col, preferred_element_type=jnp.float32)
    tot_col = jnp.dot(lincl, ps_col, preferred_element_type=jnp.float32)

    CH = 256
    rloc = lax.broadcasted_iota(jnp.int32, (CH, CH), 0)
    cloc = lax.broadcasted_iota(jnp.int32, (CH, CH), 1)
    ltri = (cloc < rloc).astype(jnp.float32)
    carry = jnp.zeros((1, E), jnp.float32)
    ranks = []
    for k in range(T // CH):
        oh_k = onehot[k * CH:(k + 1) * CH]
        cs_k = jnp.dot(ltri, oh_k, preferred_element_type=jnp.float32) + carry
        ranks.append(jnp.sum(cs_k * oh_k, axis=1, keepdims=True))
        carry = carry + jnp.sum(oh_k, axis=0, keepdims=True)
    rank = jnp.concatenate(ranks, axis=0)

    po_t = jnp.dot(onehot, po_col, preferred_element_type=jnp.float32)
    tpp_i = (po_t + rank).astype(jnp.int32)
    tpp_ref[...] = tpp_i

    tvec = lax.broadcasted_iota(jnp.int32, (T, 1), 0)
    t_hi = (tvec >> 3).astype(jnp.float32)
    t_lo = (tvec & 7).astype(jnp.float32)
    p_hi = top_prob.astype(jnp.bfloat16).astype(jnp.float32)
    p_lo = top_prob - p_hi
    lhs = jnp.concatenate([t_hi, t_lo, p_hi, p_lo], axis=1)
    for k in range(Tpad // _CHUNK):
        pp_iota = lax.broadcasted_iota(jnp.int32, (T, _CHUNK), 1) + (k * _CHUNK)
        mk = (tpp_i == pp_iota).astype(jnp.float32)
        acc = lax.dot_general(lhs, mk, (((0,), (0,)), ((), ())),
                              preferred_element_type=jnp.float32)
        gi_k = acc[0:1] * 8.0 + acc[1:2]
        pr_k = acc[2:3] + acc[3:4]
        pp_row = lax.broadcasted_iota(jnp.int32, (1, _CHUNK), 1) + (k * _CHUNK)
        fill = pp_row - jnp.where(pp_row >= T, T, 0)
        gi_i = jnp.where(pr_k > 0.0, gi_k.astype(jnp.int32), fill)
        gidx_ref[:, pl.ds(k * _CHUNK, _CHUNK)] = gi_i
        prob_ref[:, pl.ds(k * _CHUNK, _CHUNK)] = pr_k

    nlane = info_ref.shape[1]
    ti = lax.broadcasted_iota(jnp.int32, (1, nlane), 1).astype(jnp.float32) * float(_BLK)
    cmp = (ti >= tot_col).astype(jnp.int32)
    te = jnp.minimum(jnp.sum(cmp, axis=0, keepdims=True), E - 1)
    total = jnp.sum(ps_col)
    used = (ti < total).astype(jnp.int32)
    info_ref[0:1, :] = te
    info_ref[1:2, :] = used

    imp = jnp.sum(probs, axis=0, keepdims=True) / float(T)
    sizes_row = jnp.sum(onehot, axis=0, keepdims=True)
    aux = float(E) * jnp.sum(imp * sizes_row / float(T))
    aux_ref[...] = jnp.zeros_like(aux_ref) + aux


def _router(x_flat, Wr, Tpad):
    T, C = x_flat.shape
    E = Wr.shape[1]
    return pl.pallas_call(
        _router_body,
        out_shape=[
            jax.ShapeDtypeStruct((T, 1), jnp.int32),
            jax.ShapeDtypeStruct((1, Tpad), jnp.int32),
            jax.ShapeDtypeStruct((1, Tpad), jnp.float32),
            jax.ShapeDtypeStruct((8, 128), jnp.int32),
            jax.ShapeDtypeStruct((1, 128), jnp.float32),
        ],
    )(x_flat, Wr)


def _ffn_body(info_ref, xp_ref, pp_ref, w1_ref, b1_ref, w2_ref, b2_ref,
              y_ref):
    i = pl.program_id(0)

    @pl.when(info_ref[1, i] == 1)
    def _():
        xb = xp_ref[...]
        h = jnp.dot(xb, w1_ref[0], preferred_element_type=jnp.float32)
        h = _gelu_exact(h + b1_ref[0])
        yb = jnp.dot(h, w2_ref[0], preferred_element_type=jnp.float32)
        y_ref[...] = (yb + b2_ref[0]) * pp_ref[0]


def _ffn(info, x_padded, prob3, W1, b1r, W2, b2r):
    Tpad, C = x_padded.shape
    E, _, H = W1.shape
    ntiles = Tpad // _BLK
    grid_spec = pltpu.PrefetchScalarGridSpec(
        num_scalar_prefetch=1,
        grid=(ntiles,),
        in_specs=[
            pl.BlockSpec((_BLK, C), lambda i, nfo: (i, 0)),
            pl.BlockSpec((1, _BLK, 1), lambda i, nfo: (i, 0, 0)),
            pl.BlockSpec((1, C, H), lambda i, nfo: (nfo[0, i], 0, 0)),
            pl.BlockSpec((1, 1, H), lambda i, nfo: (nfo[0, i], 0, 0)),
            pl.BlockSpec((1, H, C), lambda i, nfo: (nfo[0, i], 0, 0)),
            pl.BlockSpec((1, 1, C), lambda i, nfo: (nfo[0, i], 0, 0)),
        ],
        out_specs=pl.BlockSpec((_BLK, C), lambda i, nfo: (i, 0)),
    )
    return pl.pallas_call(
        _ffn_body,
        grid_spec=grid_spec,
        out_shape=jax.ShapeDtypeStruct((Tpad, C), jnp.float32),
    )(info, x_padded, prob3, W1, b1r, W2, b2r)


def _sc_gather(table, idx):
    n = idx.shape[0]
    C = table.shape[1]
    rpw = n // _NW
    mesh = plsc.VectorSubcoreMesh(core_axis_name="c", subcore_axis_name="s")

    @functools.partial(
        pl.kernel, mesh=mesh,
        out_type=jax.ShapeDtypeStruct((n, C), jnp.float32),
        scratch_types=[
            pltpu.VMEM((rpw,), jnp.int32),
            pltpu.VMEM((rpw, C), jnp.float32),
            pltpu.SemaphoreType.DMA,
        ],
    )
    def k(table_hbm, idx_hbm, out_hbm, idx_v, rows_v, sem):
        wid = lax.axis_index("s") * _NC + lax.axis_index("c")
        base = wid * rpw
        pltpu.sync_copy(idx_hbm.at[pl.ds(base, rpw)], idx_v)
        pltpu.async_copy(table_hbm.at[idx_v], rows_v, sem).wait()
        pltpu.sync_copy(rows_v, out_hbm.at[pl.ds(base, rpw)])

    return k(table, idx)


def _sc_dispatch(x_flat, gidx):
    return _sc_gather(x_flat, gidx)


def _sc_combine(y_scaled, tpp):
    return _sc_gather(y_scaled, tpp)


def kernel(x, Wr, W1, b1, W2, b2):
    b, t, c = x.shape
    e, _, hdim = W1.shape
    x_flat = x.reshape(t, c)
    Tpad = t + e * _BLK
    ntiles = Tpad // _BLK

    tpp2d, gidx2d, prob2d, info, aux2d = _router(x_flat, Wr, Tpad)
    tpp = tpp2d.reshape(t)
    gidx = gidx2d.reshape(Tpad)
    prob3 = prob2d.reshape(ntiles, _BLK, 1)

    x_padded = _sc_dispatch(x_flat, gidx)
    b1r = b1.reshape(e, 1, hdim)
    b2r = b2.reshape(e, 1, c)
    y_scaled = _ffn(info, x_padded, prob3, W1, b1r, W2, b2r)
    y_flat = _sc_combine(y_scaled, tpp)
    return y_flat.reshape(b, t, c), aux2d[0, 0]
